# baseline (speedup 1.0000x reference)
import jax
import jax.numpy as jnp
from jax import lax
from jax.experimental import pallas as pl
from jax.experimental.pallas import tpu as pltpu
from jax.experimental.pallas import tpu_sc as plsc

NUM_ACTIONS = 100000
EMBED_DIM = 64
BATCH = 4096
HIST = 50

_NW = 32
_BB = BATCH // _NW
_PAIR_STEPS = HIST // 2


def _gather_kernel(table_hbm, act_hbm, out_hbm,
                   idxbuf, idxh, offb,
                   rows_a, rows_b, out_a, out_b,
                   gsem_a, gsem_b, wsem_a, wsem_b):
    wid = lax.axis_index("s") * 2 + lax.axis_index("c")
    b0 = pl.multiple_of(wid * _BB, _BB)

    pltpu.sync_copy(act_hbm.at[:, pl.ds(b0, _BB)], idxbuf)

    iota = lax.iota(jnp.int32, 16)
    row_ids = [g * 16 + iota for g in range(8)]

    def prep(h, carry):
        for g in range(8):
            v = idxbuf[h, pl.ds(g * 16, 16)]
            idxh[h, pl.ds(g * 16, 16)] = v >> 1
            offb[h, pl.ds(g * 16, 16)] = (v & 1) << 6
        return carry

    lax.fori_loop(0, HIST, prep, 0)

    def gather(h, rows, gsem):
        pltpu.async_copy(table_hbm.at[idxh.at[h]], rows, gsem)

    def gather_wait(h, rows, gsem):
        pltpu.make_async_copy(table_hbm.at[idxh.at[h]], rows, gsem).wait()

    def transpose(h, rows, outbuf):
        offs = [offb[h, pl.ds(g * 16, 16)] for g in range(8)]
        for d in range(EMBED_DIM):
            for g in range(8):
                v = plsc.load_gather(rows, [row_ids[g], offs[g] + d])
                outbuf[d, pl.ds(g * 16, 16)] = v

    def writeback(h, outbuf, wsem):
        pltpu.async_copy(outbuf, out_hbm.at[h, :, pl.ds(b0, _BB)], wsem)

    def writeback_wait(h, outbuf, wsem):
        pltpu.make_async_copy(
            outbuf, out_hbm.at[h, :, pl.ds(b0, _BB)], wsem).wait()

    gather(0, rows_a, gsem_a)

    def body(j, carry):
        h0 = 2 * j
        h1 = h0 + 1

        @pl.when(j > 0)
        def _():
            writeback_wait(h0, out_a, wsem_a)
            writeback_wait(h0, out_b, wsem_b)

        gather_wait(h0, rows_a, gsem_a)
        gather(h1, rows_b, gsem_b)
        transpose(h0, rows_a, out_a)
        writeback(h0, out_a, wsem_a)

        gather_wait(h1, rows_b, gsem_b)

        @pl.when(j < _PAIR_STEPS - 1)
        def _():
            gather(h0 + 2, rows_a, gsem_a)

        transpose(h1, rows_b, out_b)
        writeback(h1, out_b, wsem_b)
        return carry

    lax.fori_loop(0, _PAIR_STEPS, body, 0)
    writeback_wait(0, out_a, wsem_a)
    writeback_wait(0, out_b, wsem_b)


@jax.jit
def kernel(action, table):
    act_t = action.T
    table_h = table.reshape(NUM_ACTIONS // 2, 2 * EMBED_DIM)
    mesh = plsc.VectorSubcoreMesh(core_axis_name="c", subcore_axis_name="s")
    out_t = pl.kernel(
        _gather_kernel,
        out_type=jax.ShapeDtypeStruct((HIST, EMBED_DIM, BATCH), jnp.float32),
        mesh=mesh,
        scratch_types=[
            pltpu.VMEM((HIST, _BB), jnp.int32),
            pltpu.VMEM((HIST, _BB), jnp.int32),
            pltpu.VMEM((HIST, _BB), jnp.int32),
            pltpu.VMEM((_BB, 2 * EMBED_DIM), jnp.float32),
            pltpu.VMEM((_BB, 2 * EMBED_DIM), jnp.float32),
            pltpu.VMEM((EMBED_DIM, _BB), jnp.float32),
            pltpu.VMEM((EMBED_DIM, _BB), jnp.float32),
            pltpu.SemaphoreType.DMA,
            pltpu.SemaphoreType.DMA,
            pltpu.SemaphoreType.DMA,
            pltpu.SemaphoreType.DMA,
        ],
        compiler_params=pltpu.CompilerParams(
            use_tc_tiling_on_sc=True, needs_layout_passes=False),
    )(table_h, act_t)
    return out_t.transpose(2, 0, 1)


# speedup vs baseline: 1.2748x; 1.2748x over previous
import jax
import jax.numpy as jnp
from jax import lax
from jax.experimental import pallas as pl
from jax.experimental.pallas import tpu as pltpu
from jax.experimental.pallas import tpu_sc as plsc

NUM_ACTIONS = 100000
EMBED_DIM = 64
BATCH = 4096
HIST = 50

_NW = 32
_BB = BATCH // _NW
_PAIR_STEPS = HIST // 2


def _gather_kernel(table_hbm, act_hbm, out_hbm,
                   idxbuf, idxh, offb,
                   rows_a, rows_b, out_a, out_b,
                   gsem_a, gsem_b, wsem_a, wsem_b):
    wid = lax.axis_index("s") * 2 + lax.axis_index("c")
    b0 = pl.multiple_of(wid * _BB, _BB)

    pltpu.sync_copy(act_hbm.at[:, pl.ds(b0, _BB)], idxbuf)

    iota = lax.iota(jnp.int32, 16)
    row_ids = [g * 16 + iota for g in range(8)]

    def prep(h, carry):
        for g in range(8):
            v = idxbuf[h, pl.ds(g * 16, 16)]
            idxh[h, pl.ds(g * 16, 16)] = v >> 1
            offb[h, pl.ds(g * 16, 16)] = (v & 1) << 6
        return carry

    lax.fori_loop(0, HIST, prep, 0)

    def gather(h, rows, gsem):
        pltpu.async_copy(table_hbm.at[idxh.at[h]], rows, gsem)

    def gather_wait(h, rows, gsem):
        pltpu.make_async_copy(table_hbm.at[idxh.at[h]], rows, gsem).wait()

    def transpose(h, rows, outbuf):
        offs = [offb[h, pl.ds(g * 16, 16)] for g in range(8)]
        for d0 in range(0, EMBED_DIM, 4):
            vs = [plsc.load_gather(rows, [row_ids[g], offs[g] + (d0 + k)])
                  for k in range(4) for g in range(8)]
            for k in range(4):
                for g in range(8):
                    outbuf[d0 + k, pl.ds(g * 16, 16)] = vs[k * 8 + g]

    def writeback(h, outbuf, wsem):
        pltpu.async_copy(outbuf, out_hbm.at[h, :, pl.ds(b0, _BB)], wsem)

    def writeback_wait(h, outbuf, wsem):
        pltpu.make_async_copy(
            outbuf, out_hbm.at[h, :, pl.ds(b0, _BB)], wsem).wait()

    gather(0, rows_a, gsem_a)

    def body(j, carry):
        h0 = 2 * j
        h1 = h0 + 1

        @pl.when(j > 0)
        def _():
            writeback_wait(h0, out_a, wsem_a)
            writeback_wait(h0, out_b, wsem_b)

        gather_wait(h0, rows_a, gsem_a)
        gather(h1, rows_b, gsem_b)
        transpose(h0, rows_a, out_a)
        writeback(h0, out_a, wsem_a)

        gather_wait(h1, rows_b, gsem_b)

        @pl.when(j < _PAIR_STEPS - 1)
        def _():
            gather(h0 + 2, rows_a, gsem_a)

        transpose(h1, rows_b, out_b)
        writeback(h1, out_b, wsem_b)
        return carry

    lax.fori_loop(0, _PAIR_STEPS, body, 0)
    writeback_wait(0, out_a, wsem_a)
    writeback_wait(0, out_b, wsem_b)


@jax.jit
def kernel(action, table):
    act_t = action.T
    table_h = table.reshape(NUM_ACTIONS // 2, 2 * EMBED_DIM)
    mesh = plsc.VectorSubcoreMesh(core_axis_name="c", subcore_axis_name="s")
    out_t = pl.kernel(
        _gather_kernel,
        out_type=jax.ShapeDtypeStruct((HIST, EMBED_DIM, BATCH), jnp.float32),
        mesh=mesh,
        scratch_types=[
            pltpu.VMEM((HIST, _BB), jnp.int32),
            pltpu.VMEM((HIST, _BB), jnp.int32),
            pltpu.VMEM((HIST, _BB), jnp.int32),
            pltpu.VMEM((_BB, 2 * EMBED_DIM), jnp.float32),
            pltpu.VMEM((_BB, 2 * EMBED_DIM), jnp.float32),
            pltpu.VMEM((EMBED_DIM, _BB), jnp.float32),
            pltpu.VMEM((EMBED_DIM, _BB), jnp.float32),
            pltpu.SemaphoreType.DMA,
            pltpu.SemaphoreType.DMA,
            pltpu.SemaphoreType.DMA,
            pltpu.SemaphoreType.DMA,
        ],
        compiler_params=pltpu.CompilerParams(
            use_tc_tiling_on_sc=True, needs_layout_passes=False),
    )(table_h, act_t)
    return out_t.transpose(2, 0, 1)


# speedup vs baseline: 1.2960x; 1.0166x over previous
import jax
import jax.numpy as jnp
from jax import lax
from jax.experimental import pallas as pl
from jax.experimental.pallas import tpu as pltpu
from jax.experimental.pallas import tpu_sc as plsc

NUM_ACTIONS = 100000
EMBED_DIM = 64
BATCH = 4096
HIST = 50

_NW = 32
_BB = BATCH // _NW
_PAIR_STEPS = HIST // 2


def _gather_kernel(table_hbm, act_hbm, out_hbm,
                   idxbuf, idxh, offb,
                   rows_a, rows_b, out_a, out_b,
                   gsem_a, gsem_b, wsem_a, wsem_b):
    wid = lax.axis_index("s") * 2 + lax.axis_index("c")
    b0 = pl.multiple_of(wid * _BB, _BB)

    pltpu.sync_copy(act_hbm.at[:, pl.ds(b0, _BB)], idxbuf)

    iota = lax.iota(jnp.int32, 16)
    row_ids = [g * 16 + iota for g in range(8)]

    def prep(h, carry):
        for g in range(8):
            v = idxbuf[h, pl.ds(g * 16, 16)]
            idxh[pl.ds(h * _BB + g * 16, 16)] = v >> 1
            offb[pl.ds(h * _BB + g * 16, 16)] = (v & 1) << 6
        return carry

    lax.fori_loop(0, HIST, prep, 0)

    def gather(grp, rows, gsem):
        pltpu.async_copy(
            table_hbm.at[idxh.at[pl.ds(grp * 2 * _BB, 2 * _BB)]], rows, gsem)

    def gather_wait(grp, rows, gsem):
        pltpu.make_async_copy(
            table_hbm.at[idxh.at[pl.ds(grp * 2 * _BB, 2 * _BB)]],
            rows, gsem).wait()

    def transpose(h, rows, half, outbuf):
        offs = [offb[pl.ds(h * _BB + g * 16, 16)] for g in range(8)]
        base = half * _BB

        def dstep(i, carry):
            d0 = 4 * i
            vs = [plsc.load_gather(
                      rows, [base + row_ids[g], offs[g] + (d0 + k)])
                  for k in range(4) for g in range(8)]
            for k in range(4):
                for g in range(8):
                    outbuf[d0 + k, pl.ds(g * 16, 16)] = vs[k * 8 + g]
            return carry

        lax.fori_loop(0, EMBED_DIM // 4, dstep, 0)

    def writeback(h, outbuf, wsem):
        pltpu.async_copy(outbuf, out_hbm.at[h, :, pl.ds(b0, _BB)], wsem)

    def writeback_wait(h, outbuf, wsem):
        pltpu.make_async_copy(
            outbuf, out_hbm.at[h, :, pl.ds(b0, _BB)], wsem).wait()

    n_grp = HIST // 2
    gather(0, rows_a, gsem_a)
    gather(1, rows_b, gsem_b)

    def do_group(grp, rows, gsem, wait_outbufs):
        h0 = 2 * grp
        gather_wait(grp, rows, gsem)
        if wait_outbufs:
            writeback_wait(h0, out_a, wsem_a)
        transpose(h0, rows, 0, out_a)
        writeback(h0, out_a, wsem_a)
        if wait_outbufs:
            writeback_wait(h0 + 1, out_b, wsem_b)
        transpose(h0 + 1, rows, 1, out_b)
        writeback(h0 + 1, out_b, wsem_b)

    do_group(0, rows_a, gsem_a, False)
    gather(2, rows_a, gsem_a)
    do_group(1, rows_b, gsem_b, True)
    gather(3, rows_b, gsem_b)

    def body(j, carry):
        g0 = 2 * j
        do_group(g0, rows_a, gsem_a, True)

        @pl.when(g0 + 2 < n_grp)
        def _():
            gather(g0 + 2, rows_a, gsem_a)

        do_group(g0 + 1, rows_b, gsem_b, True)

        @pl.when(g0 + 3 < n_grp)
        def _():
            gather(g0 + 3, rows_b, gsem_b)

        return carry

    lax.fori_loop(1, n_grp // 2, body, 0)
    do_group(n_grp - 1, rows_a, gsem_a, True)
    writeback_wait(HIST - 2, out_a, wsem_a)
    writeback_wait(HIST - 1, out_b, wsem_b)


@jax.jit
def kernel(action, table):
    act_t = action.T
    table_h = table.reshape(NUM_ACTIONS // 2, 2 * EMBED_DIM)
    mesh = plsc.VectorSubcoreMesh(core_axis_name="c", subcore_axis_name="s")
    out_t = pl.kernel(
        _gather_kernel,
        out_type=jax.ShapeDtypeStruct((HIST, EMBED_DIM, BATCH), jnp.float32),
        mesh=mesh,
        scratch_types=[
            pltpu.VMEM((HIST, _BB), jnp.int32),
            pltpu.VMEM((HIST * _BB,), jnp.int32),
            pltpu.VMEM((HIST * _BB,), jnp.int32),
            pltpu.VMEM((2 * _BB, 2 * EMBED_DIM), jnp.float32),
            pltpu.VMEM((2 * _BB, 2 * EMBED_DIM), jnp.float32),
            pltpu.VMEM((EMBED_DIM, _BB), jnp.float32),
            pltpu.VMEM((EMBED_DIM, _BB), jnp.float32),
            pltpu.SemaphoreType.DMA,
            pltpu.SemaphoreType.DMA,
            pltpu.SemaphoreType.DMA,
            pltpu.SemaphoreType.DMA,
        ],
        compiler_params=pltpu.CompilerParams(
            use_tc_tiling_on_sc=True, needs_layout_passes=False),
    )(table_h, act_t)
    return out_t.transpose(2, 0, 1)


# speedup vs baseline: 2.2143x; 1.7085x over previous
import jax
import jax.numpy as jnp
from jax import lax
from jax.experimental import pallas as pl
from jax.experimental.pallas import tpu as pltpu
from jax.experimental.pallas import tpu_sc as plsc

NUM_ACTIONS = 100000
EMBED_DIM = 64
BATCH = 4096
HIST = 50

_NW = 32
_BB = BATCH // _NW
_PAIR_STEPS = HIST // 2


def _gather_kernel(table_hbm, act_hbm, out_hbm,
                   idxbuf, idxh, offb,
                   rows_a, rows_b, out_a, out_b,
                   gsem_a, gsem_b, wsem_a, wsem_b):
    wid = lax.axis_index("s") * 2 + lax.axis_index("c")
    b0 = pl.multiple_of(wid * _BB, _BB)

    pltpu.sync_copy(act_hbm.at[:, pl.ds(b0, _BB)], idxbuf)

    iota = lax.iota(jnp.int32, 16)
    row_ids = [g * 16 + iota for g in range(8)]

    def prep(h, carry):
        for g in range(8):
            v = idxbuf[h, pl.ds(g * 16, 16)]
            idxh[pl.ds(h * _BB + g * 16, 16)] = v >> 1
            offb[pl.ds(h * _BB + g * 16, 16)] = (v & 1) << 6
        return carry

    lax.fori_loop(0, HIST, prep, 0)

    def gather(grp, rows, gsem):
        pltpu.async_copy(
            table_hbm.at[idxh.at[pl.ds(grp * 2 * _BB, 2 * _BB)]], rows, gsem)

    def gather_wait(grp, rows, gsem):
        pltpu.make_async_copy(
            table_hbm.at[idxh.at[pl.ds(grp * 2 * _BB, 2 * _BB)]],
            rows, gsem).wait()

    def transpose(h, rows, half, outbuf):
        offs = [offb[pl.ds(h * _BB + g * 16, 16)] for g in range(8)]
        base = half * _BB
        col_ids = [g * 16 + iota for g in range(8)]

        def jstep(jj, carry):
            j0 = 4 * jj
            for k in range(4):
                rot = (j0 + k + iota) & (EMBED_DIM - 1)
                for g in range(8):
                    v = plsc.load_gather(
                        rows, [base + row_ids[g], offs[g] + rot])
                    plsc.store_scatter(outbuf, [rot, col_ids[g]], v)
            return carry

        lax.fori_loop(0, EMBED_DIM // 4, jstep, 0)

    def writeback(h, outbuf, wsem):
        pltpu.async_copy(outbuf, out_hbm.at[h, :, pl.ds(b0, _BB)], wsem)

    def writeback_wait(h, outbuf, wsem):
        pltpu.make_async_copy(
            outbuf, out_hbm.at[h, :, pl.ds(b0, _BB)], wsem).wait()

    n_grp = HIST // 2
    gather(0, rows_a, gsem_a)
    gather(1, rows_b, gsem_b)

    def do_group(grp, rows, gsem, wait_outbufs):
        h0 = 2 * grp
        gather_wait(grp, rows, gsem)
        if wait_outbufs:
            writeback_wait(h0, out_a, wsem_a)
        transpose(h0, rows, 0, out_a)
        writeback(h0, out_a, wsem_a)
        if wait_outbufs:
            writeback_wait(h0 + 1, out_b, wsem_b)
        transpose(h0 + 1, rows, 1, out_b)
        writeback(h0 + 1, out_b, wsem_b)

    do_group(0, rows_a, gsem_a, False)
    gather(2, rows_a, gsem_a)
    do_group(1, rows_b, gsem_b, True)
    gather(3, rows_b, gsem_b)

    def body(j, carry):
        g0 = 2 * j
        do_group(g0, rows_a, gsem_a, True)

        @pl.when(g0 + 2 < n_grp)
        def _():
            gather(g0 + 2, rows_a, gsem_a)

        do_group(g0 + 1, rows_b, gsem_b, True)

        @pl.when(g0 + 3 < n_grp)
        def _():
            gather(g0 + 3, rows_b, gsem_b)

        return carry

    lax.fori_loop(1, n_grp // 2, body, 0)
    do_group(n_grp - 1, rows_a, gsem_a, True)
    writeback_wait(HIST - 2, out_a, wsem_a)
    writeback_wait(HIST - 1, out_b, wsem_b)


@jax.jit
def kernel(action, table):
    act_t = action.T
    table_h = table.reshape(NUM_ACTIONS // 2, 2 * EMBED_DIM)
    mesh = plsc.VectorSubcoreMesh(core_axis_name="c", subcore_axis_name="s")
    out_t = pl.kernel(
        _gather_kernel,
        out_type=jax.ShapeDtypeStruct((HIST, EMBED_DIM, BATCH), jnp.float32),
        mesh=mesh,
        scratch_types=[
            pltpu.VMEM((HIST, _BB), jnp.int32),
            pltpu.VMEM((HIST * _BB,), jnp.int32),
            pltpu.VMEM((HIST * _BB,), jnp.int32),
            pltpu.VMEM((2 * _BB, 2 * EMBED_DIM), jnp.float32),
            pltpu.VMEM((2 * _BB, 2 * EMBED_DIM), jnp.float32),
            pltpu.VMEM((EMBED_DIM, _BB), jnp.float32),
            pltpu.VMEM((EMBED_DIM, _BB), jnp.float32),
            pltpu.SemaphoreType.DMA,
            pltpu.SemaphoreType.DMA,
            pltpu.SemaphoreType.DMA,
            pltpu.SemaphoreType.DMA,
        ],
        compiler_params=pltpu.CompilerParams(
            use_tc_tiling_on_sc=True, needs_layout_passes=False),
    )(table_h, act_t)
    return out_t.transpose(2, 0, 1)


# speedup vs baseline: 3.0162x; 1.3622x over previous
import jax
import jax.numpy as jnp
from jax import lax
from jax.experimental import pallas as pl
from jax.experimental.pallas import tpu as pltpu
from jax.experimental.pallas import tpu_sc as plsc

NUM_ACTIONS = 100000
EMBED_DIM = 64
BATCH = 4096
HIST = 50

_NW = 32
_BB = BATCH // _NW
_PAIR_STEPS = HIST // 2


def _gather_kernel(table_hbm, act_hbm, out_hbm,
                   idxbuf, idxh, offb,
                   rows_a, rows_b, out_a, out_b,
                   gsem_a, gsem_b, wsem_a, wsem_b):
    wid = lax.axis_index("s") * 2 + lax.axis_index("c")
    b0 = pl.multiple_of(wid * _BB, _BB)

    pltpu.sync_copy(act_hbm.at[:, pl.ds(b0, _BB)], idxbuf)

    iota = lax.iota(jnp.int32, 16)
    row_ids = [g * 16 + iota for g in range(8)]

    def prep(h, carry):
        for g in range(8):
            v = idxbuf[h, pl.ds(g * 16, 16)]
            idxh[pl.ds(h * _BB + g * 16, 16)] = v >> 1
            offb[pl.ds(h * _BB + g * 16, 16)] = (v & 1) << 6
        return carry

    lax.fori_loop(0, HIST, prep, 0)

    def gather(grp, rows, gsem):
        pltpu.async_copy(
            table_hbm.at[idxh.at[pl.ds(grp * 2 * _BB, 2 * _BB)]], rows, gsem)

    def gather_wait(grp, rows, gsem):
        pltpu.make_async_copy(
            table_hbm.at[idxh.at[pl.ds(grp * 2 * _BB, 2 * _BB)]],
            rows, gsem).wait()

    def transpose(h, rows, half, outbuf):
        offs = [offb[pl.ds(h * _BB + g * 16, 16)] for g in range(8)]
        base = half * _BB
        col_ids = [g * 16 + iota for g in range(8)]

        def jstep(jj, carry):
            j0 = 4 * jj
            for k in range(4):
                rot = (j0 + k + iota) & (EMBED_DIM - 1)
                vs = [plsc.load_gather(
                          rows, [base + row_ids[g], offs[g] + rot])
                      for g in range(8)]
                for g in range(8):
                    plsc.store_scatter(outbuf, [rot, col_ids[g]], vs[g])
            return carry

        lax.fori_loop(0, EMBED_DIM // 4, jstep, 0)

    def writeback(h, outbuf, wsem):
        pltpu.async_copy(outbuf, out_hbm.at[h, :, pl.ds(b0, _BB)], wsem)

    def writeback_wait(h, outbuf, wsem):
        pltpu.make_async_copy(
            outbuf, out_hbm.at[h, :, pl.ds(b0, _BB)], wsem).wait()

    n_grp = HIST // 2
    gather(0, rows_a, gsem_a)
    gather(1, rows_b, gsem_b)

    def do_group(grp, rows, gsem, wait_outbufs):
        h0 = 2 * grp
        gather_wait(grp, rows, gsem)
        if wait_outbufs:
            writeback_wait(h0, out_a, wsem_a)
        transpose(h0, rows, 0, out_a)
        writeback(h0, out_a, wsem_a)
        if wait_outbufs:
            writeback_wait(h0 + 1, out_b, wsem_b)
        transpose(h0 + 1, rows, 1, out_b)
        writeback(h0 + 1, out_b, wsem_b)

    do_group(0, rows_a, gsem_a, False)
    gather(2, rows_a, gsem_a)
    do_group(1, rows_b, gsem_b, True)
    gather(3, rows_b, gsem_b)

    def body(j, carry):
        g0 = 2 * j
        do_group(g0, rows_a, gsem_a, True)

        @pl.when(g0 + 2 < n_grp)
        def _():
            gather(g0 + 2, rows_a, gsem_a)

        do_group(g0 + 1, rows_b, gsem_b, True)

        @pl.when(g0 + 3 < n_grp)
        def _():
            gather(g0 + 3, rows_b, gsem_b)

        return carry

    lax.fori_loop(1, n_grp // 2, body, 0)
    do_group(n_grp - 1, rows_a, gsem_a, True)
    writeback_wait(HIST - 2, out_a, wsem_a)
    writeback_wait(HIST - 1, out_b, wsem_b)


@jax.jit
def kernel(action, table):
    act_t = action.T
    table_h = table.reshape(NUM_ACTIONS // 2, 2 * EMBED_DIM)
    mesh = plsc.VectorSubcoreMesh(core_axis_name="c", subcore_axis_name="s")
    out_t = pl.kernel(
        _gather_kernel,
        out_type=jax.ShapeDtypeStruct((HIST, EMBED_DIM, BATCH), jnp.float32),
        mesh=mesh,
        scratch_types=[
            pltpu.VMEM((HIST, _BB), jnp.int32),
            pltpu.VMEM((HIST * _BB,), jnp.int32),
            pltpu.VMEM((HIST * _BB,), jnp.int32),
            pltpu.VMEM((2 * _BB, 2 * EMBED_DIM), jnp.float32),
            pltpu.VMEM((2 * _BB, 2 * EMBED_DIM), jnp.float32),
            pltpu.VMEM((EMBED_DIM, _BB), jnp.float32),
            pltpu.VMEM((EMBED_DIM, _BB), jnp.float32),
            pltpu.SemaphoreType.DMA,
            pltpu.SemaphoreType.DMA,
            pltpu.SemaphoreType.DMA,
            pltpu.SemaphoreType.DMA,
        ],
        compiler_params=pltpu.CompilerParams(
            use_tc_tiling_on_sc=True, needs_layout_passes=False),
    )(table_h, act_t)
    return out_t.transpose(2, 0, 1)
